# baseline (speedup 1.0000x reference)
import functools

import jax
import jax.numpy as jnp
from jax import lax
from jax.experimental import pallas as pl
from jax.experimental.pallas import tpu as pltpu
from jax.experimental.pallas import tpu_sc as plsc

NE = 8
NTOT = 12
EPT = 16
ROWT = 256
NT = 40
ROWS = (NT + 1) * ROWT
DUMP = NT * ROWT
T = 512


def _a1_body_v3(x_ref, gw_ref, d0_ref, d1_ref, w_ref, cntv_ref,
                cv_ref, e0s, e1s, r0s, r1s):
    j = pl.program_id(0)
    nstep = pl.num_programs(0) - 1

    @pl.when(j == 0)
    def _init():
        cv_ref[...] = jnp.zeros_like(cv_ref)

    @pl.when(j < nstep)
    def _route():
        xb = x_ref[...]
        logits = lax.dot_general(xb, gw_ref[...], (((1,), (1,)), ((), ())),
                                 preferred_element_type=jnp.float32)
        lane = lax.broadcasted_iota(jnp.int32, logits.shape, 1)
        logits = jnp.where(lane < NTOT, logits, -jnp.inf)
        m1 = jnp.max(logits, axis=1, keepdims=True)
        i1 = jnp.min(jnp.where(logits == m1, lane, NTOT), axis=1, keepdims=True)
        masked = jnp.where(lane == i1, -jnp.inf, logits)
        m2 = jnp.max(masked, axis=1, keepdims=True)
        i2 = jnp.min(jnp.where(masked == m2, lane, NTOT), axis=1, keepdims=True)
        wa = jax.nn.sigmoid(m1 - m2)
        wb = 1.0 - wa

        oh0 = (lane == i1).astype(jnp.float32)
        oh1 = (lane == i2).astype(jnp.float32)
        tot = oh0 + oh1
        r_iota = lax.broadcasted_iota(jnp.int32, (T, T), 0)
        c_iota = lax.broadcasted_iota(jnp.int32, (T, T), 1)
        tril = (c_iota < r_iota).astype(jnp.float32)
        excl = lax.dot_general(tril, tot, (((1,), (0,)), ((), ())),
                               preferred_element_type=jnp.float32)
        cv = cv_ref[...]
        rank0 = (jnp.sum(excl * oh0, axis=1, keepdims=True)
                 + lax.dot_general(oh0, cv, (((1,), (1,)), ((), ())),
                                   preferred_element_type=jnp.float32))
        rank1 = (jnp.sum(excl * oh1, axis=1, keepdims=True)
                 + lax.dot_general(oh1, cv, (((1,), (1,)), ((), ())),
                                   preferred_element_type=jnp.float32))
        sl = pl.ds(j * T, T)
        e0s[sl, :] = i1
        e1s[sl, :] = i2
        r0s[sl, :] = rank0.astype(jnp.int32)
        r1s[sl, :] = rank1.astype(jnp.int32)
        wlane = lax.broadcasted_iota(jnp.int32, (T, 3 * EPT), 1)
        w0b = jnp.where(i1 < NE, wa, 0.0)
        w1b = jnp.where(i2 < NE, wb, 0.0)
        idwb = jnp.where(i1 >= NE, wa, 0.0) + jnp.where(i2 >= NE, wb, 0.0)
        w_ref[...] = jnp.where(
            wlane < EPT, w0b, jnp.where(wlane < 2 * EPT, w1b, idwb))
        cv_ref[...] = cv + jnp.sum(tot, axis=0, keepdims=True)

    @pl.when(j == nstep)
    def _finalize():
        lane1 = lax.broadcasted_iota(jnp.int32, (1, EPT), 1)
        cv = cv_ref[...]
        padded = jnp.where(lane1 < NE,
                           jnp.ceil(cv / ROWT) * ROWT, 0.0)
        rk = lax.broadcasted_iota(jnp.int32, (EPT, EPT), 0)
        ce = lax.broadcasted_iota(jnp.int32, (EPT, EPT), 1)
        m16 = (rk < ce).astype(jnp.float32)
        offv = lax.dot_general(padded, m16, (((1,), (0,)), ((), ())),
                               preferred_element_type=jnp.float32)
        lane = lane1
        for e_s, r_s, d_ref in ((e0s, r0s, d0_ref), (e1s, r1s, d1_ref)):
            e = e_s[...]
            ohs = (e == lane).astype(jnp.float32)
            off = lax.dot_general(ohs, offv, (((1,), (1,)), ((), ())),
                                  preferred_element_type=jnp.float32)
            d_ref[...] = jnp.where(e < NE, r_s[...] + off.astype(jnp.int32),
                                   DUMP)
        cntv_ref[...] = cv.astype(jnp.int32)


def _router_meta(xf, gw, N, D):
    nstep = N // T
    d0, d1, wpack, cntv = pl.pallas_call(
        _a1_body_v3,
        grid=(nstep + 1,),
        in_specs=[
            pl.BlockSpec((T, D), lambda j, m=nstep - 1: (jnp.minimum(j, m), 0)),
            pl.BlockSpec((EPT, D), lambda j: (0, 0)),
        ],
        out_specs=[
            pl.BlockSpec((N, 1), lambda j: (0, 0)),
            pl.BlockSpec((N, 1), lambda j: (0, 0)),
            pl.BlockSpec((T, 3 * EPT),
                         lambda j, m=nstep - 1: (jnp.minimum(j, m), 0)),
            pl.BlockSpec((1, EPT), lambda j: (0, 0)),
        ],
        out_shape=[
            jax.ShapeDtypeStruct((N, 1), jnp.int32),
            jax.ShapeDtypeStruct((N, 1), jnp.int32),
            jax.ShapeDtypeStruct((N, 3 * EPT), jnp.float32),
            jax.ShapeDtypeStruct((1, EPT), jnp.int32),
        ],
        scratch_shapes=[
            pltpu.VMEM((1, EPT), jnp.float32),
            pltpu.VMEM((N, 1), jnp.int32),
            pltpu.VMEM((N, 1), jnp.int32),
            pltpu.VMEM((N, 1), jnp.int32),
            pltpu.VMEM((N, 1), jnp.int32),
        ],
    )(xf, gw)
    return d0, d1, wpack, cntv.reshape(EPT)


def _ntiles(cnt_ref):
    nt = 0
    for k in range(NE):
        nt = nt + (cnt_ref[k] + (ROWT - 1)) // ROWT
    return nt


def _tclamp(j, cnt_ref):
    return jnp.minimum(j, jnp.maximum(_ntiles(cnt_ref) - 1, 0))


def _texp(j, cnt_ref):
    jc = _tclamp(j, cnt_ref)
    run = 0
    te = 0
    for k in range(NE):
        run = run + (cnt_ref[k] + (ROWT - 1)) // ROWT
        te = te + jnp.where(jc >= run, 1, 0)
    return jnp.minimum(te, NE - 1)


NFC = 4


def _c_body(cnt_ref, rows_ref, w1c_ref, w2_ref, o_ref, h_scr):
    j = pl.program_id(0)
    c = pl.program_id(1)
    fc = w1c_ref.shape[2]

    @pl.when((j < NT) & (j == _tclamp(j, cnt_ref)))
    def _fwd():
        hc = lax.dot_general(rows_ref[...], w1c_ref[0, 0],
                             (((1,), (1,)), ((), ())),
                             preferred_element_type=jnp.float32)
        h_scr[c] = hc * jax.nn.sigmoid(hc)

        @pl.when(c == NFC - 1)
        def _proj():
            acc = jnp.zeros_like(o_ref)
            for cc in range(NFC):
                acc += lax.dot_general(
                    h_scr[cc], w2_ref[0, :, cc * fc:(cc + 1) * fc],
                    (((1,), (1,)), ((), ())),
                    preferred_element_type=jnp.float32)
            o_ref[...] = acc

    @pl.when((j == NT) & (c == NFC - 1))
    def _zero():
        o_ref[...] = jnp.zeros_like(o_ref)


def _ffn_grouped(counts, buf_in, W1, W2, D, DFF):
    fc = DFF // NFC
    w1r = W1.reshape(W1.shape[0], NFC, fc, D)

    def w1_map(j, c, cnt):
        cc = jnp.where(j <= _tclamp(j, cnt), c, NFC - 1)
        return (_texp(j, cnt), cc, 0, 0)

    buf_out = pl.pallas_call(
        _c_body,
        grid_spec=pltpu.PrefetchScalarGridSpec(
            num_scalar_prefetch=1,
            grid=(NT + 1, NFC),
            in_specs=[
                pl.BlockSpec((ROWT, D), lambda j, c, cnt: (_tclamp(j, cnt), 0)),
                pl.BlockSpec((1, 1, fc, D), w1_map),
                pl.BlockSpec((1, D, DFF), lambda j, c, cnt: (_texp(j, cnt), 0, 0)),
            ],
            out_specs=pl.BlockSpec(
                (ROWT, D),
                lambda j, c, cnt: (jnp.where(j == NT, NT, _tclamp(j, cnt)), 0)),
            scratch_shapes=[pltpu.VMEM((NFC, ROWT, DFF // NFC), jnp.float32)],
        ),
        out_shape=jax.ShapeDtypeStruct((ROWS, D), jnp.float32),
    )(counts, buf_in, w1r, W2)
    return buf_out


SCH = 32
CCH = 16


def _sc_scatter(xf, d0, d1, N, D):
    info = plsc.get_sparse_core_info()
    NC, NS = info.num_cores, info.num_subcores
    NW = NC * NS
    per = N // NW
    nb = per // SCH
    mesh = plsc.VectorSubcoreMesh(core_axis_name="c", subcore_axis_name="s")

    @functools.partial(
        pl.kernel,
        out_type=jax.ShapeDtypeStruct((ROWS, D), jnp.float32),
        mesh=mesh,
        scratch_types=[
            [pltpu.VMEM((SCH, D), jnp.float32)] * 2,
            [pltpu.VMEM((SCH,), jnp.int32)] * 2,
            [pltpu.VMEM((SCH,), jnp.int32)] * 2,
            [pltpu.SemaphoreType.DMA] * 2,
            [pltpu.SemaphoreType.DMA] * 2,
        ],
    )
    def scatter_k(x_hbm, d0_hbm, d1_hbm, buf_hbm, xr, i0, i1, sl, ss):
        wid = lax.axis_index("s") * NC + lax.axis_index("c")

        def loads(b):
            p = b % 2
            base = wid * per + b * SCH
            cx = pltpu.async_copy(x_hbm.at[pl.ds(base, SCH)], xr[p], sl[p])
            c0 = pltpu.async_copy(d0_hbm.at[pl.ds(base, SCH)], i0[p], sl[p])
            c1 = pltpu.async_copy(d1_hbm.at[pl.ds(base, SCH)], i1[p], sl[p])
            return (cx, c0, c1)

        ld = {0: loads(0)}
        st = {}
        for b in range(nb):
            p = b % 2
            for c in ld[b]:
                c.wait()
            s0 = pltpu.async_copy(xr[p], buf_hbm.at[i0[p]], ss[p])
            s1 = pltpu.async_copy(xr[p], buf_hbm.at[i1[p]], ss[p])
            st[b] = (s0, s1)
            if b + 1 < nb:
                if b - 1 >= 0:
                    for s in st[b - 1]:
                        s.wait()
                ld[b + 1] = loads(b + 1)
        if nb >= 2:
            for s in st[nb - 2]:
                s.wait()
        for s in st[nb - 1]:
            s.wait()

    return scatter_k(xf, d0, d1)


def _sc_combine(xf, buf_out, d0, d1, wpack, N, D):
    info = plsc.get_sparse_core_info()
    NC, NS = info.num_cores, info.num_subcores
    NW = NC * NS
    per = N // NW
    nb = per // CCH
    nlc = D // 16
    mesh = plsc.VectorSubcoreMesh(core_axis_name="c", subcore_axis_name="s")

    @functools.partial(
        pl.kernel,
        out_type=jax.ShapeDtypeStruct((N, D), jnp.float32),
        mesh=mesh,
        scratch_types=[
            [pltpu.VMEM((CCH, D), jnp.float32)] * 2,
            [pltpu.VMEM((CCH, D), jnp.float32)] * 2,
            [pltpu.VMEM((CCH, D), jnp.float32)] * 2,
            pltpu.VMEM((CCH, D), jnp.float32),
            [pltpu.VMEM((CCH,), jnp.int32)] * 2,
            [pltpu.VMEM((CCH,), jnp.int32)] * 2,
            [pltpu.VMEM((CCH, 3 * EPT), jnp.float32)] * 2,
            [pltpu.SemaphoreType.DMA] * 2,
            [pltpu.SemaphoreType.DMA] * 2,
        ],
    )
    def combine_k(x_hbm, buf_hbm, p0_hbm, p1_hbm, w_hbm, out_hbm,
                  xr, g0, g1, orr, i0, i1, wr, sa, sg):
        wid = lax.axis_index("s") * NC + lax.axis_index("c")

        def loads(b):
            p = b % 2
            base = wid * per + b * CCH
            ci0 = pltpu.async_copy(p0_hbm.at[pl.ds(base, CCH)], i0[p], sa[p])
            ci1 = pltpu.async_copy(p1_hbm.at[pl.ds(base, CCH)], i1[p], sa[p])
            cx = pltpu.async_copy(x_hbm.at[pl.ds(base, CCH)], xr[p], sa[p])
            cw = pltpu.async_copy(w_hbm.at[pl.ds(base, CCH)], wr[p], sa[p])
            return (ci0, ci1, cx, cw)

        def gathers(b, lds):
            p = b % 2
            lds[0].wait()
            lds[1].wait()
            cg0 = pltpu.async_copy(buf_hbm.at[i0[p]], g0[p], sg[p])
            cg1 = pltpu.async_copy(buf_hbm.at[i1[p]], g1[p], sg[p])
            return (lds[2], lds[3], cg0, cg1)

        def compute(b, pend):
            p = b % 2
            for c in pend:
                c.wait()
            base = wid * per + b * CCH

            def tok(i, acc):
                w0v = wr[p][i, 0:16]
                w1v = wr[p][i, 16:32]
                wiv = wr[p][i, 32:48]

                @plsc.parallel_loop(0, nlc, unroll=8)
                def _lc(c):
                    s = pl.ds(c * 16, 16)
                    orr[i, s] = (wiv * xr[p][i, s] + w0v * g0[p][i, s]
                                 + w1v * g1[p][i, s])
                return acc

            lax.fori_loop(0, CCH, tok, 0)
            pltpu.sync_copy(orr, out_hbm.at[pl.ds(base, CCH)])

        gt = {}
        ld = {0: loads(0)}
        if nb > 1:
            ld[1] = loads(1)
        gt[0] = gathers(0, ld[0])
        for b in range(nb):
            if b + 1 < nb:
                gt[b + 1] = gathers(b + 1, ld[b + 1])
            compute(b, gt[b])
            if b + 2 < nb:
                ld[b + 2] = loads(b + 2)

    return combine_k(xf, buf_out, d0, d1, wpack)


def kernel(x, gate_W, W1, W2):
    B, S, D = x.shape
    num_real, DFF, _ = W1.shape
    N = B * S
    xf = x.reshape(N, D)
    gw = jnp.zeros((EPT, D), gate_W.dtype).at[:NTOT].set(gate_W)

    d0, d1, wpack, counts = _router_meta(xf, gw, N, D)
    d0f = d0.reshape(N)
    d1f = d1.reshape(N)
    buf_in = _sc_scatter(xf, d0f, d1f, N, D)
    buf_out = _ffn_grouped(counts, buf_in, W1, W2, D, DFF)
    out = _sc_combine(xf, buf_out, d0f, d1f, wpack, N, D)
    return out.reshape(B, S, D)


# speedup vs baseline: 1.0611x; 1.0611x over previous
import functools

import jax
import jax.numpy as jnp
from jax import lax
from jax.experimental import pallas as pl
from jax.experimental.pallas import tpu as pltpu
from jax.experimental.pallas import tpu_sc as plsc

NE = 8
NTOT = 12
EPT = 16
ROWT = 256
NT = 40
ROWS = (NT + 1) * ROWT
DUMP = NT * ROWT
T = 512


def _a1_body_v3(x_ref, gw_ref, d0_ref, d1_ref, w_ref, cntv_ref,
                cv_ref, e0s, e1s, r0s, r1s):
    j = pl.program_id(0)
    nstep = pl.num_programs(0) - 1

    @pl.when(j == 0)
    def _init():
        cv_ref[...] = jnp.zeros_like(cv_ref)

    @pl.when(j < nstep)
    def _route():
        xb = x_ref[...]
        logits = lax.dot_general(xb, gw_ref[...], (((1,), (1,)), ((), ())),
                                 preferred_element_type=jnp.float32)
        lane = lax.broadcasted_iota(jnp.int32, logits.shape, 1)
        logits = jnp.where(lane < NTOT, logits, -jnp.inf)
        m1 = jnp.max(logits, axis=1, keepdims=True)
        i1 = jnp.min(jnp.where(logits == m1, lane, NTOT), axis=1, keepdims=True)
        masked = jnp.where(lane == i1, -jnp.inf, logits)
        m2 = jnp.max(masked, axis=1, keepdims=True)
        i2 = jnp.min(jnp.where(masked == m2, lane, NTOT), axis=1, keepdims=True)
        wa = jax.nn.sigmoid(m1 - m2)
        wb = 1.0 - wa

        oh0 = (lane == i1).astype(jnp.float32)
        oh1 = (lane == i2).astype(jnp.float32)
        tot = oh0 + oh1
        r_iota = lax.broadcasted_iota(jnp.int32, (T, T), 0)
        c_iota = lax.broadcasted_iota(jnp.int32, (T, T), 1)
        tril = (c_iota < r_iota).astype(jnp.float32)
        excl = lax.dot_general(tril, tot, (((1,), (0,)), ((), ())),
                               preferred_element_type=jnp.float32)
        cv = cv_ref[...]
        rank0 = (jnp.sum(excl * oh0, axis=1, keepdims=True)
                 + lax.dot_general(oh0, cv, (((1,), (1,)), ((), ())),
                                   preferred_element_type=jnp.float32))
        rank1 = (jnp.sum(excl * oh1, axis=1, keepdims=True)
                 + lax.dot_general(oh1, cv, (((1,), (1,)), ((), ())),
                                   preferred_element_type=jnp.float32))
        sl = pl.ds(j * T, T)
        e0s[sl, :] = i1
        e1s[sl, :] = i2
        r0s[sl, :] = rank0.astype(jnp.int32)
        r1s[sl, :] = rank1.astype(jnp.int32)
        wlane = lax.broadcasted_iota(jnp.int32, (T, 3 * EPT), 1)
        w0b = jnp.where(i1 < NE, wa, 0.0)
        w1b = jnp.where(i2 < NE, wb, 0.0)
        idwb = jnp.where(i1 >= NE, wa, 0.0) + jnp.where(i2 >= NE, wb, 0.0)
        w_ref[...] = jnp.where(
            wlane < EPT, w0b, jnp.where(wlane < 2 * EPT, w1b, idwb))
        cv_ref[...] = cv + jnp.sum(tot, axis=0, keepdims=True)

    @pl.when(j == nstep)
    def _finalize():
        lane1 = lax.broadcasted_iota(jnp.int32, (1, EPT), 1)
        cv = cv_ref[...]
        padded = jnp.where(lane1 < NE,
                           jnp.ceil(cv / ROWT) * ROWT, 0.0)
        rk = lax.broadcasted_iota(jnp.int32, (EPT, EPT), 0)
        ce = lax.broadcasted_iota(jnp.int32, (EPT, EPT), 1)
        m16 = (rk < ce).astype(jnp.float32)
        offv = lax.dot_general(padded, m16, (((1,), (0,)), ((), ())),
                               preferred_element_type=jnp.float32)
        lane = lane1
        for e_s, r_s, d_ref in ((e0s, r0s, d0_ref), (e1s, r1s, d1_ref)):
            e = e_s[...]
            ohs = (e == lane).astype(jnp.float32)
            off = lax.dot_general(ohs, offv, (((1,), (1,)), ((), ())),
                                  preferred_element_type=jnp.float32)
            d_ref[...] = jnp.where(e < NE, r_s[...] + off.astype(jnp.int32),
                                   DUMP)
        cntv_ref[...] = cv.astype(jnp.int32)


def _router_meta(xf, gw, N, D):
    nstep = N // T
    d0, d1, wpack, cntv = pl.pallas_call(
        _a1_body_v3,
        grid=(nstep + 1,),
        in_specs=[
            pl.BlockSpec((T, D), lambda j, m=nstep - 1: (jnp.minimum(j, m), 0)),
            pl.BlockSpec((EPT, D), lambda j: (0, 0)),
        ],
        out_specs=[
            pl.BlockSpec((N, 1), lambda j: (0, 0)),
            pl.BlockSpec((N, 1), lambda j: (0, 0)),
            pl.BlockSpec((T, 3 * EPT),
                         lambda j, m=nstep - 1: (jnp.minimum(j, m), 0)),
            pl.BlockSpec((1, EPT), lambda j: (0, 0)),
        ],
        out_shape=[
            jax.ShapeDtypeStruct((N, 1), jnp.int32),
            jax.ShapeDtypeStruct((N, 1), jnp.int32),
            jax.ShapeDtypeStruct((N, 3 * EPT), jnp.float32),
            jax.ShapeDtypeStruct((1, EPT), jnp.int32),
        ],
        scratch_shapes=[
            pltpu.VMEM((1, EPT), jnp.float32),
            pltpu.VMEM((N, 1), jnp.int32),
            pltpu.VMEM((N, 1), jnp.int32),
            pltpu.VMEM((N, 1), jnp.int32),
            pltpu.VMEM((N, 1), jnp.int32),
        ],
    )(xf, gw)
    return d0, d1, wpack, cntv.reshape(EPT)


def _ntiles(cnt_ref):
    nt = 0
    for k in range(NE):
        nt = nt + (cnt_ref[k] + (ROWT - 1)) // ROWT
    return nt


def _tclamp(j, cnt_ref):
    return jnp.minimum(j, jnp.maximum(_ntiles(cnt_ref) - 1, 0))


def _texp(j, cnt_ref):
    jc = _tclamp(j, cnt_ref)
    run = 0
    te = 0
    for k in range(NE):
        run = run + (cnt_ref[k] + (ROWT - 1)) // ROWT
        te = te + jnp.where(jc >= run, 1, 0)
    return jnp.minimum(te, NE - 1)


def _c1_body(cnt_ref, rows_ref, w1_ref, h_ref):
    j = pl.program_id(0)

    @pl.when(j == _tclamp(j, cnt_ref))
    def _():
        h = lax.dot_general(rows_ref[...], w1_ref[0], (((1,), (1,)), ((), ())),
                            preferred_element_type=jnp.float32)
        h_ref[...] = h * jax.nn.sigmoid(h)


def _c2_body(cnt_ref, h_ref, w2_ref, o_ref):
    j = pl.program_id(0)

    @pl.when(j == NT)
    def _zero():
        o_ref[...] = jnp.zeros_like(o_ref)

    @pl.when((j < NT) & (j == _tclamp(j, cnt_ref)))
    def _():
        o_ref[...] = lax.dot_general(h_ref[...], w2_ref[0],
                                     (((1,), (1,)), ((), ())),
                                     preferred_element_type=jnp.float32)


def _ffn_grouped(counts, buf_in, W1, W2, D, DFF):
    h = pl.pallas_call(
        _c1_body,
        grid_spec=pltpu.PrefetchScalarGridSpec(
            num_scalar_prefetch=1,
            grid=(NT,),
            in_specs=[
                pl.BlockSpec((ROWT, D), lambda j, c: (_tclamp(j, c), 0)),
                pl.BlockSpec((1, DFF, D), lambda j, c: (_texp(j, c), 0, 0)),
            ],
            out_specs=pl.BlockSpec((ROWT, DFF), lambda j, c: (_tclamp(j, c), 0)),
        ),
        out_shape=jax.ShapeDtypeStruct((NT * ROWT, DFF), jnp.float32),
    )(counts, buf_in, W1)

    buf_out = pl.pallas_call(
        _c2_body,
        grid_spec=pltpu.PrefetchScalarGridSpec(
            num_scalar_prefetch=1,
            grid=(NT + 1,),
            in_specs=[
                pl.BlockSpec((ROWT, DFF), lambda j, c: (_tclamp(j, c), 0)),
                pl.BlockSpec((1, D, DFF), lambda j, c: (_texp(j, c), 0, 0)),
            ],
            out_specs=pl.BlockSpec(
                (ROWT, D),
                lambda j, c: (jnp.where(j == NT, NT, _tclamp(j, c)), 0)),
        ),
        out_shape=jax.ShapeDtypeStruct((ROWS, D), jnp.float32),
    )(counts, h, W2)
    return buf_out


SCH = 32
CCH = 16


def _sc_scatter(xf, d0, d1, N, D):
    info = plsc.get_sparse_core_info()
    NC, NS = info.num_cores, info.num_subcores
    NW = NC * NS
    per = N // NW
    nb = per // SCH
    mesh = plsc.VectorSubcoreMesh(core_axis_name="c", subcore_axis_name="s")

    @functools.partial(
        pl.kernel,
        out_type=jax.ShapeDtypeStruct((ROWS, D), jnp.float32),
        mesh=mesh,
        scratch_types=[
            [pltpu.VMEM((SCH, D), jnp.float32)] * 2,
            [pltpu.VMEM((SCH,), jnp.int32)] * 2,
            [pltpu.VMEM((SCH,), jnp.int32)] * 2,
            [pltpu.SemaphoreType.DMA] * 2,
            [pltpu.SemaphoreType.DMA] * 2,
        ],
    )
    def scatter_k(x_hbm, d0_hbm, d1_hbm, buf_hbm, xr, i0, i1, sl, ss):
        wid = lax.axis_index("s") * NC + lax.axis_index("c")

        def loads(b):
            p = b % 2
            base = wid * per + b * SCH
            cx = pltpu.async_copy(x_hbm.at[pl.ds(base, SCH)], xr[p], sl[p])
            c0 = pltpu.async_copy(d0_hbm.at[pl.ds(base, SCH)], i0[p], sl[p])
            c1 = pltpu.async_copy(d1_hbm.at[pl.ds(base, SCH)], i1[p], sl[p])
            return (cx, c0, c1)

        ld = {0: loads(0)}
        st = {}
        for b in range(nb):
            p = b % 2
            for c in ld[b]:
                c.wait()
            s0 = pltpu.async_copy(xr[p], buf_hbm.at[i0[p]], ss[p])
            s1 = pltpu.async_copy(xr[p], buf_hbm.at[i1[p]], ss[p])
            st[b] = (s0, s1)
            if b + 1 < nb:
                if b - 1 >= 0:
                    for s in st[b - 1]:
                        s.wait()
                ld[b + 1] = loads(b + 1)
        if nb >= 2:
            for s in st[nb - 2]:
                s.wait()
        for s in st[nb - 1]:
            s.wait()

    return scatter_k(xf, d0, d1)


def _sc_combine(xf, buf_out, d0, d1, wpack, N, D):
    info = plsc.get_sparse_core_info()
    NC, NS = info.num_cores, info.num_subcores
    NW = NC * NS
    per = N // NW
    nb = per // CCH
    nlc = D // 16
    mesh = plsc.VectorSubcoreMesh(core_axis_name="c", subcore_axis_name="s")

    @functools.partial(
        pl.kernel,
        out_type=jax.ShapeDtypeStruct((N, D), jnp.float32),
        mesh=mesh,
        scratch_types=[
            [pltpu.VMEM((CCH, D), jnp.float32)] * 2,
            [pltpu.VMEM((CCH, D), jnp.float32)] * 2,
            [pltpu.VMEM((CCH, D), jnp.float32)] * 2,
            pltpu.VMEM((CCH, D), jnp.float32),
            [pltpu.VMEM((CCH,), jnp.int32)] * 2,
            [pltpu.VMEM((CCH,), jnp.int32)] * 2,
            [pltpu.VMEM((CCH, 3 * EPT), jnp.float32)] * 2,
            [pltpu.SemaphoreType.DMA] * 2,
            [pltpu.SemaphoreType.DMA] * 2,
        ],
    )
    def combine_k(x_hbm, buf_hbm, p0_hbm, p1_hbm, w_hbm, out_hbm,
                  xr, g0, g1, orr, i0, i1, wr, sa, sg):
        wid = lax.axis_index("s") * NC + lax.axis_index("c")

        def loads(b):
            p = b % 2
            base = wid * per + b * CCH
            ci0 = pltpu.async_copy(p0_hbm.at[pl.ds(base, CCH)], i0[p], sa[p])
            ci1 = pltpu.async_copy(p1_hbm.at[pl.ds(base, CCH)], i1[p], sa[p])
            cx = pltpu.async_copy(x_hbm.at[pl.ds(base, CCH)], xr[p], sa[p])
            cw = pltpu.async_copy(w_hbm.at[pl.ds(base, CCH)], wr[p], sa[p])
            return (ci0, ci1, cx, cw)

        def gathers(b, lds):
            p = b % 2
            lds[0].wait()
            lds[1].wait()
            cg0 = pltpu.async_copy(buf_hbm.at[i0[p]], g0[p], sg[p])
            cg1 = pltpu.async_copy(buf_hbm.at[i1[p]], g1[p], sg[p])
            return (lds[2], lds[3], cg0, cg1)

        def compute(b, pend):
            p = b % 2
            for c in pend:
                c.wait()
            base = wid * per + b * CCH

            def tok(i, acc):
                w0v = wr[p][i, 0:16]
                w1v = wr[p][i, 16:32]
                wiv = wr[p][i, 32:48]

                @plsc.parallel_loop(0, nlc, unroll=8)
                def _lc(c):
                    s = pl.ds(c * 16, 16)
                    orr[i, s] = (wiv * xr[p][i, s] + w0v * g0[p][i, s]
                                 + w1v * g1[p][i, s])
                return acc

            lax.fori_loop(0, CCH, tok, 0)
            pltpu.sync_copy(orr, out_hbm.at[pl.ds(base, CCH)])

        gt = {}
        ld = {0: loads(0)}
        if nb > 1:
            ld[1] = loads(1)
        gt[0] = gathers(0, ld[0])
        for b in range(nb):
            if b + 1 < nb:
                gt[b + 1] = gathers(b + 1, ld[b + 1])
            compute(b, gt[b])
            if b + 2 < nb:
                ld[b + 2] = loads(b + 2)

    return combine_k(xf, buf_out, d0, d1, wpack)


def kernel(x, gate_W, W1, W2):
    B, S, D = x.shape
    num_real, DFF, _ = W1.shape
    N = B * S
    xf = x.reshape(N, D)
    gw = jnp.zeros((EPT, D), gate_W.dtype).at[:NTOT].set(gate_W)

    d0, d1, wpack, counts = _router_meta(xf, gw, N, D)
    d0f = d0.reshape(N)
    d1f = d1.reshape(N)
    buf_in = _sc_scatter(xf, d0f, d1f, N, D)
    buf_out = _ffn_grouped(counts, buf_in, W1, W2, D, DFF)
    out = _sc_combine(xf, buf_out, d0f, d1f, wpack, N, D)
    return out.reshape(B, S, D)
